# Initial kernel scaffold; baseline (speedup 1.0000x reference)
import jax
import jax.numpy as jnp
from jax.experimental import pallas as pl


def kernel(x_operation, x_machine, x_job, ei_om_src, ei_om_dst, ei_mo_src, ei_mo_dst, ei_oo_src, ei_oo_dst, ei_jo_src, ei_jo_dst, ei_oj_src, ei_oj_dst, vp_operation, vp_machine, vp_job, params):
    raise NotImplementedError("write your pallas kernel here")


# speedup vs baseline: 1.0000x; 1.0000x over previous
import jax
import jax.numpy as jnp
from jax.experimental import pallas as pl

NODE_TYPES = ['operation', 'machine', 'job']
N_NODES = {'operation': 50000, 'machine': 500, 'job': 2000}
EDGE_TYPES = [('operation', 'machine', 'om'), ('machine', 'operation', 'mo'),
              ('operation', 'operation', 'oo'), ('job', 'operation', 'jo'),
              ('operation', 'job', 'oj')]
H = 64
L = 3


def _bn(x, g, b):
    m = jnp.mean(x, axis=0, keepdims=True)
    v = jnp.var(x, axis=0, keepdims=True)
    return g * (x - m) / jnp.sqrt(v + 1e-5) + b


def kernel(x_operation, x_machine, x_job, ei_om_src, ei_om_dst, ei_mo_src, ei_mo_dst, ei_oo_src, ei_oo_dst, ei_jo_src, ei_jo_dst, ei_oj_src, ei_oj_dst, vp_operation, vp_machine, vp_job, params):
    p = params
    xs = {'operation': x_operation, 'machine': x_machine, 'job': x_job}
    ei = {'om': (ei_om_src, ei_om_dst), 'mo': (ei_mo_src, ei_mo_dst),
          'oo': (ei_oo_src, ei_oo_dst), 'jo': (ei_jo_src, ei_jo_dst),
          'oj': (ei_oj_src, ei_oj_dst)}
    x = {}
    for nt in NODE_TYPES:
        lin = xs[nt] @ p['enc_%s_Wl' % nt] + p['enc_%s_bl' % nt]
        per = jnp.sin(xs[nt] @ p['enc_%s_Wp' % nt] + p['enc_%s_bp' % nt])
        x[nt] = jnp.concatenate([lin, per], axis=1)
    residual = None
    for l in range(L):
        out = {nt: jnp.zeros((N_NODES[nt], H), jnp.float32) for nt in NODE_TYPES}
        for src, dst, name in EDGE_TYPES:
            s, d = ei[name]
            aggr = jax.ops.segment_sum(x[src][s], d, num_segments=N_NODES[dst])
            pre = 'conv%d_%s_' % (l, name)
            h = x[dst] + aggr
            h = h @ p[pre + 'W1'] + p[pre + 'b1']
            h = jax.nn.relu(_bn(h, p[pre + 'g1'], p[pre + 'be1']))
            h = h @ p[pre + 'W2'] + p[pre + 'b2']
            out[dst] = out[dst] + h
        if residual is not None:
            out = {nt: out[nt] + residual[nt] for nt in NODE_TYPES}
        residual = out
        x = out
    feats = jnp.concatenate([x['operation'][vp_operation], x['machine'][vp_machine], x['job'][vp_job]], axis=1)
    h = feats @ p['s_W1'] + p['s_b1']
    h = jax.nn.relu(_bn(h, p['s_g1'], p['s_be1']))
    h = h @ p['s_W2'] + p['s_b2']
    h = jax.nn.relu(_bn(h, p['s_g2'], p['s_be2']))
    h = h @ p['s_W3'] + p['s_b3']
    return h[:, 0]


# speedup vs baseline: 2.7486x; 2.7486x over previous
import functools

import jax
import jax.numpy as jnp
from jax import lax
from jax.experimental import pallas as pl
from jax.experimental.pallas import tpu as pltpu
from jax.experimental.pallas import tpu_sc as plsc

NC, NS, LANES = 2, 16, 16
NW = NC * NS
BLK = 128
KB_SPLIT = 2
KB_SMALL = 8

N_OP, N_MACH, N_JOB = 50000, 500, 2000
HALF_OP = N_OP // 2
APAD_OP = 25088
H = 64
L = 3


def _sc_mesh():
    return plsc.VectorSubcoreMesh(
        core_axis_name="c", subcore_axis_name="s",
        num_cores=NC, num_subcores=NS)


def _zero_vmem_rows(ref, nrows, width):
    zv = jnp.zeros((LANES,), jnp.float32)
    for r in range(nrows):
        for j in range(width // LANES):
            ref[r, pl.ds(j * LANES, LANES)] = zv


def _segsum_sc(table, src_idx, dst_idx, n_dst, split):
    e_tot = src_idx.shape[0] * BLK
    w = table.shape[1]
    if split:
        kb = KB_SPLIT
        apad = APAD_OP
        nsup = e_tot // (kb * BLK * NS)
    else:
        kb = KB_SMALL
        apad = ((n_dst + 1 + 127) // 128) * 128
        nsup = e_tot // (kb * BLK * NW)
    zrows = apad // NS

    def body(table_h, src_h, dst_h, out_h, idx_s, idx_d, rows, zbuf, accum,
             gsem, ssem):
        c = lax.axis_index("c")
        s = lax.axis_index("s")
        wid = c * NS + s
        _zero_vmem_rows(zbuf, LANES, w)
        for r in range(zrows // LANES):
            pltpu.sync_copy(zbuf, accum.at[pl.ds(s * zrows + r * LANES, LANES)])
        plsc.subcore_barrier()

        half = jnp.int32(HALF_OP)
        base_c = c.astype(jnp.int32) * half

        def sup_body(i, carry):
            if split:
                base = (s * nsup + i) * kb
            else:
                base = (wid * nsup + i) * kb
            pltpu.sync_copy(src_h.at[pl.ds(base, kb)], idx_s)
            pltpu.sync_copy(dst_h.at[pl.ds(base, kb)], idx_d)
            if split:
                for j in range(kb):
                    for q in range(BLK // LANES):
                        v = idx_d[j, pl.ds(q * LANES, LANES)]
                        loc = v - base_c
                        oob = (loc < 0) | (loc >= half)
                        idx_d[j, pl.ds(q * LANES, LANES)] = jnp.where(
                            oob, half, loc)
            cps = [pltpu.async_copy(table_h.at[idx_s.at[j]],
                                    rows.at[pl.ds(j * BLK, BLK)], gsem)
                   for j in range(kb)]
            for cp in cps:
                cp.wait()
            cps = [pltpu.async_copy(rows.at[pl.ds(j * BLK, BLK)],
                                    accum.at[idx_d.at[j]], ssem, add=True)
                   for j in range(kb)]
            for cp in cps:
                cp.wait()
            return carry

        lax.fori_loop(0, nsup, sup_body, 0)
        plsc.subcore_barrier()
        pltpu.sync_copy(accum.at[pl.ds(s * zrows, zrows)],
                        out_h.at[c, pl.ds(s * zrows, zrows)])

    fn = pl.kernel(
        body,
        out_type=jax.ShapeDtypeStruct((2, apad, w), jnp.float32),
        mesh=_sc_mesh(),
        scratch_types=[
            pltpu.VMEM((kb, BLK), jnp.int32),
            pltpu.VMEM((kb, BLK), jnp.int32),
            pltpu.VMEM((kb * BLK, w), jnp.float32),
            pltpu.VMEM((LANES, w), jnp.float32),
            pltpu.VMEM_SHARED((apad, w), jnp.float32),
            pltpu.SemaphoreType.DMA,
            pltpu.SemaphoreType.DMA,
        ],
        compiler_params=pltpu.CompilerParams(use_tc_tiling_on_sc=False),
    )
    return fn(table, src_idx, dst_idx)


def _pad_edges(src, dst, pad_dst, granule):
    e = src.shape[0]
    ep = ((e + granule - 1) // granule) * granule
    if ep != e:
        src = jnp.concatenate([src, jnp.zeros((ep - e,), jnp.int32)])
        dst = jnp.concatenate(
            [dst, jnp.full((ep - e,), pad_dst, jnp.int32)])
    return src.reshape(ep // BLK, BLK), dst.reshape(ep // BLK, BLK)


def _segsum_small(table, src, dst, n_dst):
    src, dst = _pad_edges(src, dst, n_dst, KB_SMALL * BLK * NW)
    out = _segsum_sc(table, src, dst, n_dst, split=False)
    return (out[0] + out[1])[:n_dst]


def _segsum_op(table, src, dst):
    src, dst = _pad_edges(src, dst, N_OP, KB_SPLIT * BLK * NS)
    out = _segsum_sc(table, src, dst, N_OP, split=True)
    return jnp.concatenate([out[0, :HALF_OP], out[1, :HALF_OP]], axis=0)


def _bn(x, g, b):
    m = jnp.mean(x, axis=0, keepdims=True)
    v = jnp.var(x, axis=0, keepdims=True)
    return g * (x - m) / jnp.sqrt(v + 1e-5) + b


def kernel(x_operation, x_machine, x_job, ei_om_src, ei_om_dst, ei_mo_src,
           ei_mo_dst, ei_oo_src, ei_oo_dst, ei_jo_src, ei_jo_dst, ei_oj_src,
           ei_oj_dst, vp_operation, vp_machine, vp_job, params):
    p = params
    n_nodes = {'operation': N_OP, 'machine': N_MACH, 'job': N_JOB}
    xs = {'operation': x_operation, 'machine': x_machine, 'job': x_job}
    ei = {'om': (ei_om_src, ei_om_dst), 'mo': (ei_mo_src, ei_mo_dst),
          'oo': (ei_oo_src, ei_oo_dst), 'jo': (ei_jo_src, ei_jo_dst),
          'oj': (ei_oj_src, ei_oj_dst)}
    edge_types = [('operation', 'machine', 'om'), ('machine', 'operation', 'mo'),
                  ('operation', 'operation', 'oo'), ('job', 'operation', 'jo'),
                  ('operation', 'job', 'oj')]
    x = {}
    for nt in ['operation', 'machine', 'job']:
        lin = xs[nt] @ p['enc_%s_Wl' % nt] + p['enc_%s_bl' % nt]
        per = jnp.sin(xs[nt] @ p['enc_%s_Wp' % nt] + p['enc_%s_bp' % nt])
        x[nt] = jnp.concatenate([lin, per], axis=1)
    residual = None
    for l in range(L):
        out = {nt: jnp.zeros((n_nodes[nt], H), jnp.float32)
               for nt in ['operation', 'machine', 'job']}
        for src_t, dst_t, name in edge_types:
            s, d = ei[name]
            if dst_t == 'operation':
                aggr = _segsum_op(x[src_t], s, d)
            else:
                aggr = _segsum_small(x[src_t], s, d, n_nodes[dst_t])
            pre = 'conv%d_%s_' % (l, name)
            h = x[dst_t] + aggr
            h = h @ p[pre + 'W1'] + p[pre + 'b1']
            h = jax.nn.relu(_bn(h, p[pre + 'g1'], p[pre + 'be1']))
            h = h @ p[pre + 'W2'] + p[pre + 'b2']
            out[dst_t] = out[dst_t] + h
        if residual is not None:
            out = {nt: out[nt] + residual[nt] for nt in out}
        residual = out
        x = out
    feats = jnp.concatenate([x['operation'][vp_operation],
                             x['machine'][vp_machine],
                             x['job'][vp_job]], axis=1)
    h = feats @ p['s_W1'] + p['s_b1']
    h = jax.nn.relu(_bn(h, p['s_g1'], p['s_be1']))
    h = h @ p['s_W2'] + p['s_b2']
    h = jax.nn.relu(_bn(h, p['s_g2'], p['s_be2']))
    h = h @ p['s_W3'] + p['s_b3']
    return h[:, 0]
